# baseline (speedup 1.0000x reference)
import functools

import jax
import jax.numpy as jnp
from jax import lax
from jax.experimental import pallas as pl
from jax.experimental.pallas import tpu as pltpu
from jax.experimental.pallas import tpu_sc as plsc


def _proj_body(e_ref, w_ref, b_ref, p_ref):
    p_ref[...] = (
        jnp.dot(e_ref[...], w_ref[...], preferred_element_type=jnp.float32)
        + b_ref[...]
    )


def _project_table(table, wt, bias2d):
    v, d = table.shape
    c = wt.shape[1]
    vb = 8000
    assert v % vb == 0
    return pl.pallas_call(
        _proj_body,
        grid=(v // vb,),
        in_specs=[
            pl.BlockSpec((vb, d), lambda i: (i, 0)),
            pl.BlockSpec((d, c), lambda i: (0, 0)),
            pl.BlockSpec((1, c), lambda i: (0, 0)),
        ],
        out_specs=pl.BlockSpec((vb, c), lambda i: (i, 0)),
        out_shape=jax.ShapeDtypeStruct((v, c), jnp.float32),
    )(table, wt, bias2d)


def _sc_gather(ptable, idx_flat, chunk):
    info = plsc.get_sparse_core_info()
    nw = info.num_cores * info.num_subcores
    n = idx_flat.shape[0]
    c = ptable.shape[1]
    b_per_w = n // nw
    assert b_per_w * nw == n and b_per_w % chunk == 0
    n_chunks = b_per_w // chunk
    mesh = plsc.VectorSubcoreMesh(core_axis_name="c", subcore_axis_name="s")

    @functools.partial(
        pl.kernel,
        mesh=mesh,
        out_type=jax.ShapeDtypeStruct((n, c), jnp.float32),
        scratch_types=[
            pltpu.VMEM((chunk,), jnp.int32),
            pltpu.VMEM((chunk, c), jnp.float32),
            pltpu.SemaphoreType.DMA,
        ],
        compiler_params=pltpu.CompilerParams(use_tc_tiling_on_sc=False),
    )
    def k(p_hbm, idx_hbm, out_hbm, idx_v, rows_v, sem):
        wid = lax.axis_index("s") * info.num_cores + lax.axis_index("c")
        w_base = wid * b_per_w

        def body(g, carry):
            base = w_base + g * chunk
            pltpu.sync_copy(idx_hbm.at[pl.ds(base, chunk)], idx_v)
            pltpu.async_copy(p_hbm.at[idx_v], rows_v, sem).wait()
            pltpu.sync_copy(rows_v, out_hbm.at[pl.ds(base, chunk)])
            return carry

        lax.fori_loop(0, n_chunks, body, 0)

    return k(ptable, idx_flat)


def kernel(input, embedding_weight, fc_weight, fc_bias):
    b, l = input.shape
    num_classes = fc_weight.shape[0]
    wt = fc_weight.T
    bias2d = fc_bias.reshape(1, num_classes)
    ptable = _project_table(embedding_weight, wt, bias2d)
    idx_flat = input.reshape(-1)
    out_flat = _sc_gather(ptable, idx_flat, chunk=12800)
    return out_flat.reshape(b, l, num_classes)


# speedup vs baseline: 15.9034x; 15.9034x over previous
import functools

import jax
import jax.numpy as jnp
from jax import lax
from jax.experimental import pallas as pl
from jax.experimental.pallas import tpu as pltpu
from jax.experimental.pallas import tpu_sc as plsc


def _proj_body(e_ref, w_ref, b_ref, out_ref):
    p = (
        jnp.dot(w_ref[...], e_ref[...], preferred_element_type=jnp.float32)
        + b_ref[...]
    )
    pb = p.astype(jnp.bfloat16)
    u = lax.bitcast_convert_type(pb, jnp.uint16).astype(jnp.uint32)
    packed = u[0] | (u[1] << 16)
    out_ref[...] = packed.reshape(out_ref.shape)


def _project_pack(table_t, w, bias2d, rows):
    d, v = table_t.shape
    cv = 49152
    grid = (v + cv - 1) // cv
    assert rows * 128 == grid * cv and rows % 8 == 0
    return pl.pallas_call(
        _proj_body,
        grid=(grid,),
        in_specs=[
            pl.BlockSpec((d, cv), lambda i: (0, i)),
            pl.BlockSpec((2, d), lambda i: (0, 0)),
            pl.BlockSpec((2, 1), lambda i: (0, 0)),
        ],
        out_specs=pl.BlockSpec((cv // 128, 128), lambda i: (i, 0)),
        out_shape=jax.ShapeDtypeStruct((rows, 128), jnp.uint32),
    )(table_t, w, bias2d)


def _sc_lookup(packed_flat, idx_t, cb):
    info = plsc.get_sparse_core_info()
    nw = info.num_cores * info.num_subcores
    vp = packed_flat.shape[0]
    n_l, n_b = idx_t.shape
    n_bb = n_b // cb
    n_tasks = n_l * n_bb
    tasks_per_w = n_tasks // nw
    assert n_bb * cb == n_b and tasks_per_w * nw == n_tasks
    mesh = plsc.VectorSubcoreMesh(core_axis_name="c", subcore_axis_name="s")

    nt = cb // 128

    @functools.partial(
        pl.kernel,
        mesh=mesh,
        out_type=jax.ShapeDtypeStruct((n_l, n_b // 128, 2, 128), jnp.float32),
        scratch_types=[
            pltpu.VMEM_SHARED((vp,), jnp.uint32),
            pltpu.VMEM((2, cb), jnp.int32),
            pltpu.VMEM((2, cb), jnp.uint32),
            pltpu.VMEM((2, nt, 2, 128), jnp.float32),
            pltpu.SemaphoreType.DMA((2,)),
            pltpu.SemaphoreType.DMA((2,)),
        ],
        compiler_params=pltpu.CompilerParams(use_tc_tiling_on_sc=False),
    )
    def k(p_hbm, idx_hbm, out_hbm, shared, idx_v, words_v, inter_v, sem_g, sem_o):
        sid = lax.axis_index("s")
        wid = sid * info.num_cores + lax.axis_index("c")
        t0 = wid * tasks_per_w
        tpw = tasks_per_w

        def task_lb(t):
            return t // n_bb, (t % n_bb) * cb

        @pl.when(sid == 0)
        def _load_table():
            pltpu.sync_copy(p_hbm, shared)

        plsc.subcore_barrier()

        def fire_gather(s):
            pltpu.async_copy(shared.at[idx_v.at[s]], words_v.at[s], sem_g.at[s])

        def wait_gather(s):
            pltpu.make_async_copy(
                shared.at[idx_v.at[s]], words_v.at[s], sem_g.at[s]
            ).wait()

        l0, b00 = task_lb(t0)
        pltpu.sync_copy(idx_hbm.at[l0, pl.ds(b00, cb)], idx_v.at[0])
        fire_gather(0)

        def step(kk, carry):
            t = t0 + kk
            slot = lax.rem(kk, 2)
            nslot = lax.rem(kk + 1, 2)
            l, b0 = task_lb(t)

            @pl.when(kk >= 2)
            def _drain_prev():
                l2, b02 = task_lb(t - 2)
                pltpu.make_async_copy(
                    inter_v.at[slot],
                    out_hbm.at[l2, pl.ds(b02 // 128, nt)],
                    sem_o.at[slot],
                ).wait()

            @pl.when(kk + 1 < tpw)
            def _prefetch_next():
                l1, b01 = task_lb(t + 1)
                pltpu.sync_copy(idx_hbm.at[l1, pl.ds(b01, cb)], idx_v.at[nslot])
                fire_gather(nslot)

            wait_gather(slot)

            def unpack_bt(bt, c):
                for jj in range(8):
                    off = bt * 128 + jj * 16
                    w1 = words_v[slot, pl.ds(off, 16)]
                    sixteen = jnp.full((16,), 16, jnp.uint32)
                    inter_v[slot, bt, 0, pl.ds(jj * 16, 16)] = (
                        lax.bitcast_convert_type(
                            lax.shift_left(w1, sixteen), jnp.float32
                        )
                    )
                    w2 = words_v[slot, pl.ds(off, 16)]
                    himask = jnp.full((16,), 0xFFFF0000, jnp.uint32)
                    inter_v[slot, bt, 1, pl.ds(jj * 16, 16)] = (
                        lax.bitcast_convert_type(w2 & himask, jnp.float32)
                    )
                return c

            lax.fori_loop(0, nt, unpack_bt, 0)
            pltpu.async_copy(
                inter_v.at[slot], out_hbm.at[l, pl.ds(b0 // 128, nt)], sem_o.at[slot]
            )
            return carry

        lax.fori_loop(0, tpw, step, 0)

        for d in (2, 1):
            if tpw >= d:
                td = t0 + tpw - d
                ld, b0d = task_lb(td)
                sd = (tpw - d) % 2
                pltpu.make_async_copy(
                    inter_v.at[sd],
                    out_hbm.at[ld, pl.ds(b0d // 128, nt)],
                    sem_o.at[sd],
                ).wait()

    return k(packed_flat, idx_t)


def kernel(input, embedding_weight, fc_weight, fc_bias):
    b, l = input.shape
    num_classes = fc_weight.shape[0]
    assert num_classes == 2
    bias2d = fc_bias.reshape(num_classes, 1)
    packed = _project_pack(embedding_weight.T, fc_weight, bias2d, rows=8064)
    packed_flat = jnp.reshape(packed, (-1,))
    out4 = _sc_lookup(packed_flat, input.T, cb=4096)
    out_p = jnp.transpose(out4, (1, 3, 0, 2))
    return jnp.reshape(out_p, (b, l, num_classes))
